# baseline (speedup 1.0000x reference)
import jax
import jax.numpy as jnp
from jax import lax
from jax.experimental import pallas as pl
from jax.experimental.pallas import tpu as pltpu
from jax.experimental.pallas import tpu_sc as plsc

_MARGIN = 1.0
_DIM = 64
_NV = _DIM // 16
_CHUNK = 128


def _sc_repack(entT):
    E = entT.shape[1]
    info = plsc.get_sparse_core_info()
    nc, ns = info.num_cores, info.num_subcores
    nw = nc * ns
    SPAN = 256
    nspan = E // SPAN
    tail = E - nspan * SPAN
    nmain = (nspan // nw) * nw
    nextra = nspan - nmain
    npipe = nmain // nw // 2
    orow = SPAN // 2
    mesh = plsc.VectorSubcoreMesh(core_axis_name="c", subcore_axis_name="s")

    def body(src, out2, b0, b1, o0, o1, bt, si0, si1, so0, so1):
        wid = lax.axis_index("s") * nc + lax.axis_index("c")
        lanes = lax.iota(jnp.int32, 16)

        def start_in(t, buf, sem):
            c0 = pl.multiple_of((wid + nw * t) * SPAN, SPAN)
            for bi in range(_DIM // 8):
                pltpu.async_copy(
                    src.at[pl.ds(8 * bi, 8), pl.ds(c0, SPAN)],
                    buf.at[pl.ds(8 * bi, 8), :], sem)

        def wait_in(buf, sem):
            for bi in range(_DIM // 8):
                pltpu.make_async_copy(
                    src.at[pl.ds(0, 8), pl.ds(0, SPAN)],
                    buf.at[pl.ds(8 * bi, 8), :], sem).wait()

        def wait_out(buf, sem):
            pltpu.make_async_copy(
                src.at[:, pl.ds(0, SPAN)], buf, sem).wait()

        def transpose_span(binv, boutv, nent):
            def grp(g, c2):
                evec = lanes + g * 16
                rvec = evec // 2
                cbase = (evec % 2) * 64
                for d in range(_DIM):
                    vals = binv[d, pl.ds(g * 16, 16)]
                    plsc.store_scatter(boutv, [rvec, cbase + d], vals)
                return c2

            lax.fori_loop(0, nent // 16, grp, 0)

        def start_out(t, boutv, sem):
            r0 = pl.multiple_of((wid + nw * t) * orow, orow)
            pltpu.async_copy(boutv, out2.at[pl.ds(r0, orow), :], sem)

        start_in(0, b0, si0)

        def pipe(tt, carry):
            t0 = 2 * tt
            start_in(t0 + 1, b1, si1)
            wait_in(b0, si0)

            @pl.when(tt > 0)
            def _():
                wait_out(o0, so0)

            transpose_span(b0, o0, SPAN)
            start_out(t0, o0, so0)

            @pl.when(tt < npipe - 1)
            def _():
                start_in(t0 + 2, b0, si0)

            wait_in(b1, si1)

            @pl.when(tt > 0)
            def _():
                wait_out(o1, so1)

            transpose_span(b1, o1, SPAN)
            start_out(t0 + 1, o1, so1)
            return carry

        lax.fori_loop(0, npipe, pipe, 0)
        wait_out(o0, so0)
        wait_out(o1, so1)

        if nextra:
            @pl.when(wid < nextra)
            def _():
                c0 = pl.multiple_of((nmain + wid) * SPAN, SPAN)
                pltpu.sync_copy(src.at[:, pl.ds(c0, SPAN)], b0)
                transpose_span(b0, o0, SPAN)
                r0 = pl.multiple_of((nmain + wid) * orow, orow)
                pltpu.sync_copy(o0, out2.at[pl.ds(r0, orow), :])

        if tail:
            @pl.when(wid == nextra)
            def _():
                pltpu.async_copy(
                    src.at[:, pl.ds(nspan * SPAN, tail)], bt, si0).wait()
                transpose_span(bt, o1, tail)
                pltpu.sync_copy(
                    o1.at[pl.ds(0, tail // 2), :],
                    out2.at[pl.ds(nspan * SPAN // 2, tail // 2), :])

    f = pl.kernel(
        body,
        out_type=jax.ShapeDtypeStruct((E // 2, 128), jnp.float32),
        mesh=mesh,
        compiler_params=pltpu.CompilerParams(
            use_tc_tiling_on_sc=True, needs_layout_passes=False),
        scratch_types=(
            [pltpu.VMEM((_DIM, SPAN), jnp.float32) for _ in range(2)]
            + [pltpu.VMEM((SPAN // 2, 128), jnp.float32) for _ in range(2)]
            + [pltpu.VMEM((_DIM, tail if tail else 1), jnp.float32)]
            + [pltpu.SemaphoreType.DMA for _ in range(4)]
        ),
    )
    return f(entT)


def _sc_partials(ent2, rel2, h_idx, r_idx, t_idx, hn_idx, tn_idx):
    B = h_idx.shape[0]
    info = plsc.get_sparse_core_info()
    nc, ns = info.num_cores, info.num_subcores
    nw = nc * ns
    per_w = B // nw
    chunk = _CHUNK if per_w % _CHUNK == 0 else per_w
    nchunk = per_w // chunk
    ngrp = chunk // 16
    orow = chunk * 16 // 128
    mesh = plsc.VectorSubcoreMesh(core_axis_name="c", subcore_axis_name="s")

    def body(ent, rel, hi_h, ri_h, ti_h, hni_h, tni_h, pos_out, neg_out,
             hi, ri, ti, hni, tni, hv, rv, tv, hnv, tnv, opos, oneg, sem):
        wid = lax.axis_index("s") * nc + lax.axis_index("c")

        def do_chunk(ci, carry):
            base = pl.multiple_of(wid * per_w + ci * chunk, chunk)
            pltpu.sync_copy(hi_h.at[pl.ds(base, chunk)], hi)
            pltpu.sync_copy(ri_h.at[pl.ds(base, chunk)], ri)
            pltpu.sync_copy(ti_h.at[pl.ds(base, chunk)], ti)
            pltpu.sync_copy(hni_h.at[pl.ds(base, chunk)], hni)
            pltpu.sync_copy(tni_h.at[pl.ds(base, chunk)], tni)

            def fire(g, c2):
                hvec = hi[pl.ds(g * 16, 16)]
                rvec = ri[pl.ds(g * 16, 16)]
                tvec = ti[pl.ds(g * 16, 16)]
                hnvec = hni[pl.ds(g * 16, 16)]
                tnvec = tni[pl.ds(g * 16, 16)]
                for j in range(16):
                    dst = g * 16 + j
                    pltpu.async_copy(
                        ent.at[pl.ds(hvec[j] // 2, 1), :],
                        hv.at[pl.ds(dst, 1), :], sem)
                    pltpu.async_copy(
                        rel.at[pl.ds(rvec[j] // 2, 1), :],
                        rv.at[pl.ds(dst, 1), :], sem)
                    pltpu.async_copy(
                        ent.at[pl.ds(tvec[j] // 2, 1), :],
                        tv.at[pl.ds(dst, 1), :], sem)
                    pltpu.async_copy(
                        ent.at[pl.ds(hnvec[j] // 2, 1), :],
                        hnv.at[pl.ds(dst, 1), :], sem)
                    pltpu.async_copy(
                        ent.at[pl.ds(tnvec[j] // 2, 1), :],
                        tnv.at[pl.ds(dst, 1), :], sem)
                return c2

            lax.fori_loop(0, ngrp, fire, 0)
            pltpu.make_async_copy(ent.at[pl.ds(0, chunk), :], hv, sem).wait()
            pltpu.make_async_copy(ent.at[pl.ds(0, chunk), :], rv, sem).wait()
            pltpu.make_async_copy(ent.at[pl.ds(0, chunk), :], tv, sem).wait()
            pltpu.make_async_copy(ent.at[pl.ds(0, chunk), :], hnv, sem).wait()
            pltpu.make_async_copy(ent.at[pl.ds(0, chunk), :], tnv, sem).wait()

            def compute(g, c2):
                hvec = hi[pl.ds(g * 16, 16)]
                rvec = ri[pl.ds(g * 16, 16)]
                tvec = ti[pl.ds(g * 16, 16)]
                hnvec = hni[pl.ds(g * 16, 16)]
                tnvec = tni[pl.ds(g * 16, 16)]
                for j in range(16):
                    i = g * 16 + j
                    oh = (hvec[j] % 2) * 64
                    orr = (rvec[j] % 2) * 64
                    ot = (tvec[j] % 2) * 64
                    ohn = (hnvec[j] % 2) * 64
                    otn = (tnvec[j] % 2) * 64
                    accp = None
                    accn = None
                    for k in range(_NV):
                        o = k * 16
                        rk = rv[i, pl.ds(orr + o, 16)]
                        d = hv[i, pl.ds(oh + o, 16)] + rk - tv[i, pl.ds(ot + o, 16)]
                        dn = (hnv[i, pl.ds(ohn + o, 16)] + rk
                              - tnv[i, pl.ds(otn + o, 16)])
                        accp = d * d if accp is None else accp + d * d
                        accn = dn * dn if accn is None else accn + dn * dn
                    opos[2 * g + j // 8, pl.ds((j % 8) * 16, 16)] = accp
                    oneg[2 * g + j // 8, pl.ds((j % 8) * 16, 16)] = accn
                return c2

            lax.fori_loop(0, ngrp, compute, 0)
            row_base = pl.multiple_of(base * 16 // 128, orow)
            pltpu.sync_copy(opos, pos_out.at[pl.ds(row_base, orow), :])
            pltpu.sync_copy(oneg, neg_out.at[pl.ds(row_base, orow), :])
            return carry

        lax.fori_loop(0, nchunk, do_chunk, 0)

    f = pl.kernel(
        body,
        out_type=(
            jax.ShapeDtypeStruct((B * 16 // 128, 128), jnp.float32),
            jax.ShapeDtypeStruct((B * 16 // 128, 128), jnp.float32),
        ),
        mesh=mesh,
        compiler_params=pltpu.CompilerParams(use_tc_tiling_on_sc=True),
        scratch_types=(
            [pltpu.VMEM((chunk,), jnp.int32) for _ in range(5)]
            + [pltpu.VMEM((chunk, 2 * _DIM), jnp.float32) for _ in range(5)]
            + [pltpu.VMEM((orow, 128), jnp.float32) for _ in range(2)]
            + [pltpu.SemaphoreType.DMA]
        ),
    )
    return f(ent2, rel2, h_idx, r_idx, t_idx, hn_idx, tn_idx)


def _tc_loss(pos_part, neg_part):
    def body(p_ref, n_ref, o_ref):
        row = lax.broadcasted_iota(jnp.int32, (128, 8), 0)
        col = lax.broadcasted_iota(jnp.int32, (128, 8), 1)
        m = jnp.where(row // 16 == col, 1.0, 0.0).astype(jnp.float32)
        ps = jnp.dot(p_ref[...], m, preferred_element_type=jnp.float32)
        ns = jnp.dot(n_ref[...], m, preferred_element_type=jnp.float32)
        v = jnp.maximum(_MARGIN + jnp.sqrt(ps) - jnp.sqrt(ns), 0.0)
        o_ref[0, 0] = jnp.sum(v) * (1.0 / 4096.0)

    out = pl.pallas_call(
        body,
        out_shape=jax.ShapeDtypeStruct((1, 1), jnp.float32),
        out_specs=pl.BlockSpec(memory_space=pltpu.SMEM),
    )(pos_part, neg_part)
    return out[0, 0]


def kernel(ent_emb, rel_emb, h_idx, r_idx, t_idx, h_neg_idx, t_neg_idx):
    ent2 = _sc_repack(ent_emb.T)
    rel2 = jnp.reshape(rel_emb, (rel_emb.shape[0] // 2, 2 * _DIM))
    pos_part, neg_part = _sc_partials(
        ent2, rel2, h_idx, r_idx, t_idx, h_neg_idx, t_neg_idx
    )
    return _tc_loss(pos_part, neg_part)


# speedup vs baseline: 1.2271x; 1.2271x over previous
import jax
import jax.numpy as jnp
from jax import lax
from jax.experimental import pallas as pl
from jax.experimental.pallas import tpu as pltpu
from jax.experimental.pallas import tpu_sc as plsc

_MARGIN = 1.0
_DIM = 64
_NV = _DIM // 16
_CHUNK = 128


def _sc_repack(entT):
    E = entT.shape[1]
    info = plsc.get_sparse_core_info()
    nc, ns = info.num_cores, info.num_subcores
    nw = nc * ns
    SPAN = 256
    nspan = E // SPAN
    tail = E - nspan * SPAN
    nmain = (nspan // nw) * nw
    nextra = nspan - nmain
    npipe = nmain // nw // 2
    orow = SPAN // 2
    mesh = plsc.VectorSubcoreMesh(core_axis_name="c", subcore_axis_name="s")

    def body(src, out2, b0, b1, o0, o1, bt, si0, si1, so0, so1):
        wid = lax.axis_index("s") * nc + lax.axis_index("c")
        lanes = lax.iota(jnp.int32, 16)

        def start_in(t, buf, sem):
            c0 = pl.multiple_of((wid + nw * t) * SPAN, SPAN)
            for bi in range(_DIM // 8):
                pltpu.async_copy(
                    src.at[pl.ds(8 * bi, 8), pl.ds(c0, SPAN)],
                    buf.at[pl.ds(8 * bi, 8), :], sem)

        def wait_in(buf, sem):
            for bi in range(_DIM // 8):
                pltpu.make_async_copy(
                    src.at[pl.ds(0, 8), pl.ds(0, SPAN)],
                    buf.at[pl.ds(8 * bi, 8), :], sem).wait()

        def wait_out(buf, sem):
            pltpu.make_async_copy(
                src.at[:, pl.ds(0, SPAN)], buf, sem).wait()

        def transpose_span(binv, boutv, nent):
            @plsc.parallel_loop(0, nent // 16, unroll=2)
            def grp(g):
                evec = lanes + g * 16
                rvec = evec // 2
                cbase = (evec % 2) * 64
                for d in range(_DIM):
                    vals = binv[d, pl.ds(g * 16, 16)]
                    plsc.store_scatter(boutv, [rvec, cbase + d], vals)

        def start_out(t, boutv, sem):
            r0 = pl.multiple_of((wid + nw * t) * orow, orow)
            pltpu.async_copy(boutv, out2.at[pl.ds(r0, orow), :], sem)

        start_in(0, b0, si0)

        def pipe(tt, carry):
            t0 = 2 * tt
            start_in(t0 + 1, b1, si1)
            wait_in(b0, si0)

            @pl.when(tt > 0)
            def _():
                wait_out(o0, so0)

            transpose_span(b0, o0, SPAN)
            start_out(t0, o0, so0)

            @pl.when(tt < npipe - 1)
            def _():
                start_in(t0 + 2, b0, si0)

            wait_in(b1, si1)

            @pl.when(tt > 0)
            def _():
                wait_out(o1, so1)

            transpose_span(b1, o1, SPAN)
            start_out(t0 + 1, o1, so1)
            return carry

        lax.fori_loop(0, npipe, pipe, 0)
        wait_out(o0, so0)
        wait_out(o1, so1)

        if nextra:
            @pl.when(wid < nextra)
            def _():
                c0 = pl.multiple_of((nmain + wid) * SPAN, SPAN)
                pltpu.sync_copy(src.at[:, pl.ds(c0, SPAN)], b0)
                transpose_span(b0, o0, SPAN)
                r0 = pl.multiple_of((nmain + wid) * orow, orow)
                pltpu.sync_copy(o0, out2.at[pl.ds(r0, orow), :])

        if tail:
            @pl.when(wid == nextra)
            def _():
                pltpu.async_copy(
                    src.at[:, pl.ds(nspan * SPAN, tail)], bt, si0).wait()
                transpose_span(bt, o1, tail)
                pltpu.sync_copy(
                    o1.at[pl.ds(0, tail // 2), :],
                    out2.at[pl.ds(nspan * SPAN // 2, tail // 2), :])

    f = pl.kernel(
        body,
        out_type=jax.ShapeDtypeStruct((E // 2, 128), jnp.float32),
        mesh=mesh,
        compiler_params=pltpu.CompilerParams(
            use_tc_tiling_on_sc=True, needs_layout_passes=False),
        scratch_types=(
            [pltpu.VMEM((_DIM, SPAN), jnp.float32) for _ in range(2)]
            + [pltpu.VMEM((SPAN // 2, 128), jnp.float32) for _ in range(2)]
            + [pltpu.VMEM((_DIM, tail if tail else 1), jnp.float32)]
            + [pltpu.SemaphoreType.DMA for _ in range(4)]
        ),
    )
    return f(entT)


def _sc_partials(ent2, rel2, h_idx, r_idx, t_idx, hn_idx, tn_idx):
    B = h_idx.shape[0]
    info = plsc.get_sparse_core_info()
    nc, ns = info.num_cores, info.num_subcores
    nw = nc * ns
    per_w = B // nw
    chunk = _CHUNK if per_w % _CHUNK == 0 else per_w
    nchunk = per_w // chunk
    ngrp = chunk // 16
    orow = chunk * 16 // 128
    mesh = plsc.VectorSubcoreMesh(core_axis_name="c", subcore_axis_name="s")

    def body(ent, rel, hi_h, ri_h, ti_h, hni_h, tni_h, pos_out, neg_out,
             hi, ri, ti, hni, tni, hv, rv, tv, hnv, tnv, opos, oneg, sem):
        wid = lax.axis_index("s") * nc + lax.axis_index("c")

        def do_chunk(ci, carry):
            base = pl.multiple_of(wid * per_w + ci * chunk, chunk)
            pltpu.sync_copy(hi_h.at[pl.ds(base, chunk)], hi)
            pltpu.sync_copy(ri_h.at[pl.ds(base, chunk)], ri)
            pltpu.sync_copy(ti_h.at[pl.ds(base, chunk)], ti)
            pltpu.sync_copy(hni_h.at[pl.ds(base, chunk)], hni)
            pltpu.sync_copy(tni_h.at[pl.ds(base, chunk)], tni)

            def fire(g, c2):
                hvec = hi[pl.ds(g * 16, 16)]
                rvec = ri[pl.ds(g * 16, 16)]
                tvec = ti[pl.ds(g * 16, 16)]
                hnvec = hni[pl.ds(g * 16, 16)]
                tnvec = tni[pl.ds(g * 16, 16)]
                for j in range(16):
                    dst = g * 16 + j
                    pltpu.async_copy(
                        ent.at[pl.ds(hvec[j] // 2, 1), :],
                        hv.at[pl.ds(dst, 1), :], sem)
                    pltpu.async_copy(
                        rel.at[pl.ds(rvec[j] // 2, 1), :],
                        rv.at[pl.ds(dst, 1), :], sem)
                    pltpu.async_copy(
                        ent.at[pl.ds(tvec[j] // 2, 1), :],
                        tv.at[pl.ds(dst, 1), :], sem)
                    pltpu.async_copy(
                        ent.at[pl.ds(hnvec[j] // 2, 1), :],
                        hnv.at[pl.ds(dst, 1), :], sem)
                    pltpu.async_copy(
                        ent.at[pl.ds(tnvec[j] // 2, 1), :],
                        tnv.at[pl.ds(dst, 1), :], sem)
                return c2

            lax.fori_loop(0, ngrp, fire, 0)
            pltpu.make_async_copy(ent.at[pl.ds(0, chunk), :], hv, sem).wait()
            pltpu.make_async_copy(ent.at[pl.ds(0, chunk), :], rv, sem).wait()
            pltpu.make_async_copy(ent.at[pl.ds(0, chunk), :], tv, sem).wait()
            pltpu.make_async_copy(ent.at[pl.ds(0, chunk), :], hnv, sem).wait()
            pltpu.make_async_copy(ent.at[pl.ds(0, chunk), :], tnv, sem).wait()

            def compute(g, c2):
                hvec = hi[pl.ds(g * 16, 16)]
                rvec = ri[pl.ds(g * 16, 16)]
                tvec = ti[pl.ds(g * 16, 16)]
                hnvec = hni[pl.ds(g * 16, 16)]
                tnvec = tni[pl.ds(g * 16, 16)]
                for j in range(16):
                    i = g * 16 + j
                    oh = (hvec[j] % 2) * 64
                    orr = (rvec[j] % 2) * 64
                    ot = (tvec[j] % 2) * 64
                    ohn = (hnvec[j] % 2) * 64
                    otn = (tnvec[j] % 2) * 64
                    accp = None
                    accn = None
                    for k in range(_NV):
                        o = k * 16
                        rk = rv[i, pl.ds(orr + o, 16)]
                        d = hv[i, pl.ds(oh + o, 16)] + rk - tv[i, pl.ds(ot + o, 16)]
                        dn = (hnv[i, pl.ds(ohn + o, 16)] + rk
                              - tnv[i, pl.ds(otn + o, 16)])
                        accp = d * d if accp is None else accp + d * d
                        accn = dn * dn if accn is None else accn + dn * dn
                    opos[2 * g + j // 8, pl.ds((j % 8) * 16, 16)] = accp
                    oneg[2 * g + j // 8, pl.ds((j % 8) * 16, 16)] = accn
                return c2

            lax.fori_loop(0, ngrp, compute, 0)
            row_base = pl.multiple_of(base * 16 // 128, orow)
            pltpu.sync_copy(opos, pos_out.at[pl.ds(row_base, orow), :])
            pltpu.sync_copy(oneg, neg_out.at[pl.ds(row_base, orow), :])
            return carry

        lax.fori_loop(0, nchunk, do_chunk, 0)

    f = pl.kernel(
        body,
        out_type=(
            jax.ShapeDtypeStruct((B * 16 // 128, 128), jnp.float32),
            jax.ShapeDtypeStruct((B * 16 // 128, 128), jnp.float32),
        ),
        mesh=mesh,
        compiler_params=pltpu.CompilerParams(use_tc_tiling_on_sc=True),
        scratch_types=(
            [pltpu.VMEM((chunk,), jnp.int32) for _ in range(5)]
            + [pltpu.VMEM((chunk, 2 * _DIM), jnp.float32) for _ in range(5)]
            + [pltpu.VMEM((orow, 128), jnp.float32) for _ in range(2)]
            + [pltpu.SemaphoreType.DMA]
        ),
    )
    return f(ent2, rel2, h_idx, r_idx, t_idx, hn_idx, tn_idx)


def _tc_loss(pos_part, neg_part):
    def body(p_ref, n_ref, o_ref):
        row = lax.broadcasted_iota(jnp.int32, (128, 8), 0)
        col = lax.broadcasted_iota(jnp.int32, (128, 8), 1)
        m = jnp.where(row // 16 == col, 1.0, 0.0).astype(jnp.float32)
        ps = jnp.dot(p_ref[...], m, preferred_element_type=jnp.float32)
        ns = jnp.dot(n_ref[...], m, preferred_element_type=jnp.float32)
        v = jnp.maximum(_MARGIN + jnp.sqrt(ps) - jnp.sqrt(ns), 0.0)
        o_ref[0, 0] = jnp.sum(v) * (1.0 / 4096.0)

    out = pl.pallas_call(
        body,
        out_shape=jax.ShapeDtypeStruct((1, 1), jnp.float32),
        out_specs=pl.BlockSpec(memory_space=pltpu.SMEM),
    )(pos_part, neg_part)
    return out[0, 0]


def kernel(ent_emb, rel_emb, h_idx, r_idx, t_idx, h_neg_idx, t_neg_idx):
    ent2 = _sc_repack(ent_emb.T)
    rel2 = jnp.reshape(rel_emb, (rel_emb.shape[0] // 2, 2 * _DIM))
    pos_part, neg_part = _sc_partials(
        ent2, rel2, h_idx, r_idx, t_idx, h_neg_idx, t_neg_idx
    )
    return _tc_loss(pos_part, neg_part)


# speedup vs baseline: 3.2729x; 2.6673x over previous
import jax
import jax.numpy as jnp
from jax import lax
from jax.experimental import pallas as pl
from jax.experimental.pallas import tpu as pltpu
from jax.experimental.pallas import tpu_sc as plsc

_MARGIN = 1.0
_DIM = 64
_NV = _DIM // 16
_CHUNK = 128


def _sc_partials(ent_emb, rel_emb, h_idx, r_idx, t_idx, hn_idx, tn_idx):
    B = h_idx.shape[0]
    info = plsc.get_sparse_core_info()
    nc, ns = info.num_cores, info.num_subcores
    nw = nc * ns
    per_w = B // nw
    chunk = _CHUNK if per_w % _CHUNK == 0 else per_w
    nchunk = per_w // chunk
    ngrp = chunk // 16
    orow = chunk * 16 // 128
    mesh = plsc.VectorSubcoreMesh(core_axis_name="c", subcore_axis_name="s")

    def body(ent, rel, hi_h, ri_h, ti_h, hni_h, tni_h, pos_out, neg_out,
             hi, ri, ti, hni, tni, hv, rv, tv, hnv, tnv, opos, oneg, sem):
        wid = lax.axis_index("s") * nc + lax.axis_index("c")

        def do_chunk(ci, carry):
            base = pl.multiple_of(wid * per_w + ci * chunk, chunk)
            pltpu.sync_copy(hi_h.at[pl.ds(base, chunk)], hi)
            pltpu.sync_copy(ri_h.at[pl.ds(base, chunk)], ri)
            pltpu.sync_copy(ti_h.at[pl.ds(base, chunk)], ti)
            pltpu.sync_copy(hni_h.at[pl.ds(base, chunk)], hni)
            pltpu.sync_copy(tni_h.at[pl.ds(base, chunk)], tni)

            def fire(g, c2):
                hvec = hi[pl.ds(g * 16, 16)]
                rvec = ri[pl.ds(g * 16, 16)]
                tvec = ti[pl.ds(g * 16, 16)]
                hnvec = hni[pl.ds(g * 16, 16)]
                tnvec = tni[pl.ds(g * 16, 16)]
                for j in range(16):
                    dst = g * 16 + j
                    pltpu.async_copy(
                        ent.at[pl.ds(hvec[j], 1), :],
                        hv.at[pl.ds(dst, 1), :], sem)
                    pltpu.async_copy(
                        rel.at[pl.ds(rvec[j], 1), :],
                        rv.at[pl.ds(dst, 1), :], sem)
                    pltpu.async_copy(
                        ent.at[pl.ds(tvec[j], 1), :],
                        tv.at[pl.ds(dst, 1), :], sem)
                    pltpu.async_copy(
                        ent.at[pl.ds(hnvec[j], 1), :],
                        hnv.at[pl.ds(dst, 1), :], sem)
                    pltpu.async_copy(
                        ent.at[pl.ds(tnvec[j], 1), :],
                        tnv.at[pl.ds(dst, 1), :], sem)
                return c2

            lax.fori_loop(0, ngrp, fire, 0)
            pltpu.make_async_copy(ent.at[pl.ds(0, chunk), :], hv, sem).wait()
            pltpu.make_async_copy(ent.at[pl.ds(0, chunk), :], rv, sem).wait()
            pltpu.make_async_copy(ent.at[pl.ds(0, chunk), :], tv, sem).wait()
            pltpu.make_async_copy(ent.at[pl.ds(0, chunk), :], hnv, sem).wait()
            pltpu.make_async_copy(ent.at[pl.ds(0, chunk), :], tnv, sem).wait()

            def compute(g, c2):
                for j in range(16):
                    i = g * 16 + j
                    accp = None
                    accn = None
                    for k in range(_NV):
                        sl = pl.ds(k * 16, 16)
                        rk = rv[i, sl]
                        d = hv[i, sl] + rk - tv[i, sl]
                        dn = hnv[i, sl] + rk - tnv[i, sl]
                        accp = d * d if accp is None else accp + d * d
                        accn = dn * dn if accn is None else accn + dn * dn
                    opos[2 * g + j // 8, pl.ds((j % 8) * 16, 16)] = accp
                    oneg[2 * g + j // 8, pl.ds((j % 8) * 16, 16)] = accn
                return c2

            lax.fori_loop(0, ngrp, compute, 0)
            row_base = pl.multiple_of(base * 16 // 128, orow)
            pltpu.sync_copy(opos, pos_out.at[pl.ds(row_base, orow), :])
            pltpu.sync_copy(oneg, neg_out.at[pl.ds(row_base, orow), :])
            return carry

        lax.fori_loop(0, nchunk, do_chunk, 0)

    f = pl.kernel(
        body,
        out_type=(
            jax.ShapeDtypeStruct((B * 16 // 128, 128), jnp.float32),
            jax.ShapeDtypeStruct((B * 16 // 128, 128), jnp.float32),
        ),
        mesh=mesh,
        compiler_params=pltpu.CompilerParams(use_tc_tiling_on_sc=True),
        scratch_types=(
            [pltpu.VMEM((chunk,), jnp.int32) for _ in range(5)]
            + [pltpu.VMEM((chunk, _DIM), jnp.float32) for _ in range(5)]
            + [pltpu.VMEM((orow, 128), jnp.float32) for _ in range(2)]
            + [pltpu.SemaphoreType.DMA]
        ),
    )
    return f(ent_emb, rel_emb, h_idx, r_idx, t_idx, hn_idx, tn_idx)


def _tc_loss(pos_part, neg_part):
    def body(p_ref, n_ref, o_ref):
        row = lax.broadcasted_iota(jnp.int32, (128, 8), 0)
        col = lax.broadcasted_iota(jnp.int32, (128, 8), 1)
        m = jnp.where(row // 16 == col, 1.0, 0.0).astype(jnp.float32)
        ps = jnp.dot(p_ref[...], m, preferred_element_type=jnp.float32)
        ns = jnp.dot(n_ref[...], m, preferred_element_type=jnp.float32)
        v = jnp.maximum(_MARGIN + jnp.sqrt(ps) - jnp.sqrt(ns), 0.0)
        o_ref[0, 0] = jnp.sum(v) * (1.0 / 4096.0)

    out = pl.pallas_call(
        body,
        out_shape=jax.ShapeDtypeStruct((1, 1), jnp.float32),
        out_specs=pl.BlockSpec(memory_space=pltpu.SMEM),
    )(pos_part, neg_part)
    return out[0, 0]


def kernel(ent_emb, rel_emb, h_idx, r_idx, t_idx, h_neg_idx, t_neg_idx):
    pos_part, neg_part = _sc_partials(
        ent_emb, rel_emb, h_idx, r_idx, t_idx, h_neg_idx, t_neg_idx
    )
    return _tc_loss(pos_part, neg_part)


# speedup vs baseline: 4.6112x; 1.4089x over previous
import jax
import jax.numpy as jnp
from jax import lax
from jax.experimental import pallas as pl
from jax.experimental.pallas import tpu as pltpu
from jax.experimental.pallas import tpu_sc as plsc

_MARGIN = 1.0
_DIM = 64
_NV = _DIM // 16
_CHUNK = 128


def _sc_partials(ent3, rel_emb, h_idx, r_idx, t_idx, hn_idx, tn_idx):
    B = h_idx.shape[0]
    info = plsc.get_sparse_core_info()
    nc, ns = info.num_cores, info.num_subcores
    nw = nc * ns
    per_w = B // nw
    chunk = _CHUNK if per_w % _CHUNK == 0 else per_w
    nchunk = per_w // chunk
    ngrp = chunk // 16
    orow = chunk * 16 // 128
    mesh = plsc.VectorSubcoreMesh(core_axis_name="c", subcore_axis_name="s")

    half = 500000

    def body(ent, rel, hi_h, ri_h, ti_h, hni_h, tni_h, pos_out, neg_out,
             hi, ri, ti, hni, tni, hv, rv, tv, hnv, tnv, opos, oneg, sem):
        wid = lax.axis_index("s") * nc + lax.axis_index("c")

        def do_chunk(ci, carry):
            base = pl.multiple_of(wid * per_w + ci * chunk, chunk)
            pltpu.sync_copy(hi_h.at[pl.ds(base, chunk)], hi)
            pltpu.sync_copy(ri_h.at[pl.ds(base, chunk)], ri)
            pltpu.sync_copy(ti_h.at[pl.ds(base, chunk)], ti)
            pltpu.sync_copy(hni_h.at[pl.ds(base, chunk)], hni)
            pltpu.sync_copy(tni_h.at[pl.ds(base, chunk)], tni)

            def fire(g, c2):
                hvec = hi[pl.ds(g * 16, 16)]
                rvec = ri[pl.ds(g * 16, 16)]
                tvec = ti[pl.ds(g * 16, 16)]
                hnvec = hni[pl.ds(g * 16, 16)]
                tnvec = tni[pl.ds(g * 16, 16)]
                for j in range(16):
                    dst = g * 16 + j
                    e0 = hvec[j]
                    pltpu.async_copy(
                        ent.at[e0 // half, pl.ds(e0 % half, 1), :],
                        hv.at[pl.ds(dst, 1), :], sem)
                    pltpu.async_copy(
                        rel.at[pl.ds(rvec[j], 1), :],
                        rv.at[pl.ds(dst, 1), :], sem)
                    e1 = tvec[j]
                    pltpu.async_copy(
                        ent.at[e1 // half, pl.ds(e1 % half, 1), :],
                        tv.at[pl.ds(dst, 1), :], sem)
                    e2 = hnvec[j]
                    pltpu.async_copy(
                        ent.at[e2 // half, pl.ds(e2 % half, 1), :],
                        hnv.at[pl.ds(dst, 1), :], sem)
                    e3 = tnvec[j]
                    pltpu.async_copy(
                        ent.at[e3 // half, pl.ds(e3 % half, 1), :],
                        tnv.at[pl.ds(dst, 1), :], sem)
                return c2

            lax.fori_loop(0, ngrp, fire, 0)
            pltpu.make_async_copy(ent.at[0, pl.ds(0, chunk), :], hv, sem).wait()
            pltpu.make_async_copy(ent.at[0, pl.ds(0, chunk), :], rv, sem).wait()
            pltpu.make_async_copy(ent.at[0, pl.ds(0, chunk), :], tv, sem).wait()
            pltpu.make_async_copy(ent.at[0, pl.ds(0, chunk), :], hnv, sem).wait()
            pltpu.make_async_copy(ent.at[0, pl.ds(0, chunk), :], tnv, sem).wait()

            def compute(g, c2):
                for j in range(16):
                    i = g * 16 + j
                    accp = None
                    accn = None
                    for k in range(_NV):
                        sl = pl.ds(k * 16, 16)
                        rk = rv[i, sl]
                        d = hv[i, sl] + rk - tv[i, sl]
                        dn = hnv[i, sl] + rk - tnv[i, sl]
                        accp = d * d if accp is None else accp + d * d
                        accn = dn * dn if accn is None else accn + dn * dn
                    opos[2 * g + j // 8, pl.ds((j % 8) * 16, 16)] = accp
                    oneg[2 * g + j // 8, pl.ds((j % 8) * 16, 16)] = accn
                return c2

            lax.fori_loop(0, ngrp, compute, 0)
            row_base = pl.multiple_of(base * 16 // 128, orow)
            pltpu.sync_copy(opos, pos_out.at[pl.ds(row_base, orow), :])
            pltpu.sync_copy(oneg, neg_out.at[pl.ds(row_base, orow), :])
            return carry

        lax.fori_loop(0, nchunk, do_chunk, 0)

    f = pl.kernel(
        body,
        out_type=(
            jax.ShapeDtypeStruct((B * 16 // 128, 128), jnp.float32),
            jax.ShapeDtypeStruct((B * 16 // 128, 128), jnp.float32),
        ),
        mesh=mesh,
        compiler_params=pltpu.CompilerParams(use_tc_tiling_on_sc=True),
        scratch_types=(
            [pltpu.VMEM((chunk,), jnp.int32) for _ in range(5)]
            + [pltpu.VMEM((chunk, _DIM), jnp.float32) for _ in range(5)]
            + [pltpu.VMEM((orow, 128), jnp.float32) for _ in range(2)]
            + [pltpu.SemaphoreType.DMA]
        ),
    )
    return f(ent3, rel_emb, h_idx, r_idx, t_idx, hn_idx, tn_idx)


def _tc_loss(pos_part, neg_part):
    def body(p_ref, n_ref, o_ref):
        row = lax.broadcasted_iota(jnp.int32, (128, 8), 0)
        col = lax.broadcasted_iota(jnp.int32, (128, 8), 1)
        m = jnp.where(row // 16 == col, 1.0, 0.0).astype(jnp.float32)
        ps = jnp.dot(p_ref[...], m, preferred_element_type=jnp.float32)
        ns = jnp.dot(n_ref[...], m, preferred_element_type=jnp.float32)
        v = jnp.maximum(_MARGIN + jnp.sqrt(ps) - jnp.sqrt(ns), 0.0)
        o_ref[0, 0] = jnp.sum(v) * (1.0 / 4096.0)

    out = pl.pallas_call(
        body,
        out_shape=jax.ShapeDtypeStruct((1, 1), jnp.float32),
        out_specs=pl.BlockSpec(memory_space=pltpu.SMEM),
    )(pos_part, neg_part)
    return out[0, 0]


def kernel(ent_emb, rel_emb, h_idx, r_idx, t_idx, h_neg_idx, t_neg_idx):
    ent3 = jnp.reshape(ent_emb, (2, ent_emb.shape[0] // 2, _DIM))
    pos_part, neg_part = _sc_partials(
        ent3, rel_emb, h_idx, r_idx, t_idx, h_neg_idx, t_neg_idx
    )
    return _tc_loss(pos_part, neg_part)
